# baseline (speedup 1.0000x reference)
import functools

import jax
import jax.numpy as jnp
from jax import lax
from jax.experimental import pallas as pl
from jax.experimental.pallas import tpu as pltpu
from jax.experimental.pallas import tpu_sc as plsc

NUM_USERS = 100000
NUM_ITEMS = 100000
EMBED_DIM = 64
BATCH = 16384

NUM_CORES = 2
NUM_SUBCORES = 16
NW = NUM_CORES * NUM_SUBCORES
BPW = BATCH // NW
LANES = 16
HALF = BPW // 2
NGROUP = HALF // LANES


def _wid():
    return lax.axis_index("s") * NUM_CORES + lax.axis_index("c")


def _issue(idx_ref, table_hbm, rows_vmem, h0, g, sem):
    i0 = g * LANES
    vec = idx_ref[pl.ds(h0 + i0, LANES)]
    for j in range(LANES):
        pltpu.async_copy(table_hbm.at[vec[j]], rows_vmem.at[i0 + j], sem)


def _drain(table_hbm, rows_vmem, sem):
    for j in range(LANES):
        pltpu.make_async_copy(table_hbm.at[0], rows_vmem.at[j], sem).wait()


def _gather_body(users_hbm, ut_hbm, stage_hbm, uidx, urows, sem0, sem1):
    base = _wid() * BPW
    pltpu.sync_copy(users_hbm.at[pl.ds(base, BPW)], uidx)

    def half(h, carry):
        h0 = h * HALF
        _issue(uidx, ut_hbm, urows, h0, 0, sem0)

        def step(p, carry):
            _issue(uidx, ut_hbm, urows, h0, 2 * p + 1, sem1)
            _drain(ut_hbm, urows, sem0)

            @pl.when(2 * p + 2 < NGROUP)
            def _():
                _issue(uidx, ut_hbm, urows, h0, 2 * p + 2, sem0)

            _drain(ut_hbm, urows, sem1)
            return carry

        lax.fori_loop(0, NGROUP // 2, step, 0)
        pltpu.sync_copy(urows, stage_hbm.at[pl.ds(base + h0, HALF)])
        return carry

    lax.fori_loop(0, 2, half, 0)


def _dot_body(items_hbm, it_hbm, stage_hbm, out_hbm,
              iidx, urows, irows, tbuf, outv, sem0, sem1, semu):
    base = _wid() * BPW
    pltpu.sync_copy(items_hbm.at[pl.ds(base, BPW)], iidx)

    col = lax.iota(jnp.int32, LANES) * LANES

    def compute_group(h0, g):
        for b in range(LANES):
            row = g * LANES + b
            p = jnp.zeros((LANES,), jnp.float32)
            for k in range(EMBED_DIM // LANES):
                u = urows[row, pl.ds(k * LANES, LANES)]
                v = irows[row, pl.ds(k * LANES, LANES)]
                p = p + u * v
            plsc.store_scatter(tbuf, [col + b], p)
        acc = jnp.zeros((LANES,), jnp.float32)
        for r in range(LANES):
            acc = acc + tbuf[pl.ds(r * LANES, LANES)]
        outv[pl.ds(h0 + g * LANES, LANES)] = acc

    def half(h, carry):
        h0 = h * HALF
        ucp = pltpu.async_copy(stage_hbm.at[pl.ds(base + h0, HALF)], urows, semu)
        _issue(iidx, it_hbm, irows, h0, 0, sem0)
        ucp.wait()

        def step(p, carry):
            _issue(iidx, it_hbm, irows, h0, 2 * p + 1, sem1)
            _drain(it_hbm, irows, sem0)
            compute_group(h0, 2 * p)

            @pl.when(2 * p + 2 < NGROUP)
            def _():
                _issue(iidx, it_hbm, irows, h0, 2 * p + 2, sem0)

            _drain(it_hbm, irows, sem1)
            compute_group(h0, 2 * p + 1)
            return carry

        lax.fori_loop(0, NGROUP // 2, step, 0)
        return carry

    lax.fori_loop(0, 2, half, 0)

    pltpu.sync_copy(outv, out_hbm.at[pl.ds(base, BPW)])


@jax.jit
def kernel(users, items, user_table, item_table):
    mesh = plsc.VectorSubcoreMesh(core_axis_name="c", subcore_axis_name="s",
                                  num_cores=NUM_CORES, num_subcores=NUM_SUBCORES)
    gather_u = functools.partial(
        pl.kernel,
        out_type=jax.ShapeDtypeStruct((BATCH, EMBED_DIM), jnp.float32),
        mesh=mesh,
        scratch_types=[
            pltpu.VMEM((BPW,), jnp.int32),
            pltpu.VMEM((HALF, EMBED_DIM), jnp.float32),
            pltpu.SemaphoreType.DMA,
            pltpu.SemaphoreType.DMA,
        ],
        compiler_params=pltpu.CompilerParams(needs_layout_passes=False),
    )(_gather_body)
    dot = functools.partial(
        pl.kernel,
        out_type=jax.ShapeDtypeStruct((BATCH,), jnp.float32),
        mesh=mesh,
        scratch_types=[
            pltpu.VMEM((BPW,), jnp.int32),
            pltpu.VMEM((HALF, EMBED_DIM), jnp.float32),
            pltpu.VMEM((HALF, EMBED_DIM), jnp.float32),
            pltpu.VMEM((LANES * LANES,), jnp.float32),
            pltpu.VMEM((BPW,), jnp.float32),
            pltpu.SemaphoreType.DMA,
            pltpu.SemaphoreType.DMA,
            pltpu.SemaphoreType.DMA,
        ],
        compiler_params=pltpu.CompilerParams(needs_layout_passes=False),
    )(_dot_body)
    stage = gather_u(users.astype(jnp.int32), user_table)
    return dot(items.astype(jnp.int32), item_table, stage)


# speedup vs baseline: 1.0222x; 1.0222x over previous
import functools

import jax
import jax.numpy as jnp
from jax import lax
from jax.experimental import pallas as pl
from jax.experimental.pallas import tpu as pltpu
from jax.experimental.pallas import tpu_sc as plsc

NUM_USERS = 100000
NUM_ITEMS = 100000
EMBED_DIM = 64
BATCH = 16384

NUM_CORES = 2
NUM_SUBCORES = 16
NW = NUM_CORES * NUM_SUBCORES
BPW = BATCH // NW
LANES = 16
HALF = BPW // 2
NGROUP = HALF // LANES


def _dot_body(users_hbm, items_hbm, ut_hbm, it_hbm, out_hbm,
              uidx, iidx, urows, irows, tbuf, outv, sem0, sem1):
    wid = lax.axis_index("s") * NUM_CORES + lax.axis_index("c")
    base = wid * BPW

    pltpu.sync_copy(users_hbm.at[pl.ds(base, BPW)], uidx)
    pltpu.sync_copy(items_hbm.at[pl.ds(base, BPW)], iidx)

    col = lax.iota(jnp.int32, LANES) * LANES

    def issue(h0, g, sem):
        i0 = g * LANES
        uvec = uidx[pl.ds(h0 + i0, LANES)]
        ivec = iidx[pl.ds(h0 + i0, LANES)]
        for j in range(LANES):
            pltpu.async_copy(ut_hbm.at[uvec[j]], urows.at[i0 + j], sem)
            pltpu.async_copy(it_hbm.at[ivec[j]], irows.at[i0 + j], sem)

    def drain_group(sem):
        for j in range(LANES):
            pltpu.make_async_copy(ut_hbm.at[0], urows.at[j], sem).wait()
            pltpu.make_async_copy(it_hbm.at[0], irows.at[j], sem).wait()

    def compute_group(h0, g):
        for b in range(LANES):
            row = g * LANES + b
            p = jnp.zeros((LANES,), jnp.float32)
            for k in range(EMBED_DIM // LANES):
                u = urows[row, pl.ds(k * LANES, LANES)]
                v = irows[row, pl.ds(k * LANES, LANES)]
                p = p + u * v
            plsc.store_scatter(tbuf, [col + b], p)
        acc = jnp.zeros((LANES,), jnp.float32)
        for r in range(LANES):
            acc = acc + tbuf[pl.ds(r * LANES, LANES)]
        outv[pl.ds(h0 + g * LANES, LANES)] = acc

    def half(h, carry):
        h0 = h * HALF
        issue(h0, 0, sem0)

        def step(p, carry):
            issue(h0, 2 * p + 1, sem1)
            drain_group(sem0)
            compute_group(h0, 2 * p)

            @pl.when(2 * p + 2 < NGROUP)
            def _():
                issue(h0, 2 * p + 2, sem0)

            drain_group(sem1)
            compute_group(h0, 2 * p + 1)
            return carry

        lax.fori_loop(0, NGROUP // 2, step, 0)
        return carry

    lax.fori_loop(0, 2, half, 0)

    pltpu.sync_copy(outv, out_hbm.at[pl.ds(base, BPW)])


@jax.jit
def kernel(users, items, user_table, item_table):
    mesh = plsc.VectorSubcoreMesh(core_axis_name="c", subcore_axis_name="s",
                                  num_cores=NUM_CORES, num_subcores=NUM_SUBCORES)
    run = functools.partial(
        pl.kernel,
        out_type=jax.ShapeDtypeStruct((BATCH,), jnp.float32),
        mesh=mesh,
        scratch_types=[
            pltpu.VMEM((BPW,), jnp.int32),
            pltpu.VMEM((BPW,), jnp.int32),
            pltpu.VMEM((HALF, EMBED_DIM), jnp.float32),
            pltpu.VMEM((HALF, EMBED_DIM), jnp.float32),
            pltpu.VMEM((LANES * LANES,), jnp.float32),
            pltpu.VMEM((BPW,), jnp.float32),
            pltpu.SemaphoreType.DMA,
            pltpu.SemaphoreType.DMA,
        ],
        compiler_params=pltpu.CompilerParams(needs_layout_passes=False),
    )(_dot_body)
    return run(users.astype(jnp.int32), items.astype(jnp.int32),
               user_table, item_table)
